# Initial kernel scaffold; baseline (speedup 1.0000x reference)
import jax
import jax.numpy as jnp
from jax.experimental import pallas as pl


def kernel(node_input, node_attr, edge_src, edge_dst, edge_attr, edge_scalars, W_sc, W_lin1, fc_W1, fc_W2, W_lin2, W_lin3):
    raise NotImplementedError("write your pallas kernel here")


# speedup vs baseline: 2.2663x; 2.2663x over previous
import functools

import numpy as np
import jax
import jax.numpy as jnp
from jax import lax
from jax.experimental import pallas as pl
from jax.experimental.pallas import tpu as pltpu
from jax.experimental.pallas import tpu_sc as plsc

N, E, D, S, H = 10000, 320000, 128, 16, 64
INV_SQRT_D = 1.0 / np.sqrt(128.0)
INV_SQRT_S = 1.0 / np.sqrt(16.0)
INV_SQRT_H = 1.0 / np.sqrt(64.0)
INV_SQRT_NN = 1.0 / np.sqrt(32.0)


def _node_body(x_ref, attr_ref, wsc_ref, w1_ref, nsc_ref, nf_ref):
    x = x_ref[...]
    s = attr_ref[...] * INV_SQRT_D
    nsc_ref[...] = jnp.dot(x, wsc_ref[...], preferred_element_type=jnp.float32) * s
    nf_ref[...] = jnp.dot(x, w1_ref[...], preferred_element_type=jnp.float32) * s


def _node_stage(node_input, node_attr, wsc, w1):
    return pl.pallas_call(
        _node_body,
        out_shape=(
            jax.ShapeDtypeStruct((N, D), jnp.float32),
            jax.ShapeDtypeStruct((N, D), jnp.float32),
        ),
    )(node_input, node_attr, wsc, w1)


_BE = 3200
_EG = E // _BE


def _edge_body(es_ref, ea_ref, w1_ref, w2_ref, wa_ref):
    h = jnp.dot(es_ref[...], w1_ref[...], preferred_element_type=jnp.float32)
    h = jax.nn.silu(h * INV_SQRT_S)
    w = jnp.dot(h, w2_ref[...], preferred_element_type=jnp.float32) * INV_SQRT_H
    wa_ref[...] = w * ea_ref[...]


def _edge_stage(edge_scalars, edge_attr, fc_W1, fc_W2):
    return pl.pallas_call(
        _edge_body,
        grid=(_EG,),
        in_specs=[
            pl.BlockSpec((_BE, S), lambda i: (i, 0)),
            pl.BlockSpec((_BE, 1), lambda i: (i, 0)),
            pl.BlockSpec((S, H), lambda i: (0, 0)),
            pl.BlockSpec((H, D), lambda i: (0, 0)),
        ],
        out_specs=pl.BlockSpec((_BE, D), lambda i: (i, 0)),
        out_shape=jax.ShapeDtypeStruct((E, D), jnp.float32),
    )(edge_scalars, edge_attr, fc_W1, fc_W2)


_NC, _NS = 2, 16
_NW = _NC * _NS
_EPW = E // _NW
_KB = 80
_NB = _EPW // _KB
_RPS = 632
_NP = _RPS * _NS


def _make_sc_stage():
    mesh = plsc.VectorSubcoreMesh(core_axis_name="c", subcore_axis_name="s")

    @functools.partial(
        pl.kernel,
        out_type=jax.ShapeDtypeStruct((_NC * _NP, D), jnp.float32),
        mesh=mesh,
        scratch_types=[
            pltpu.VMEM((_KB,), jnp.int32),
            pltpu.VMEM((_KB,), jnp.int32),
            pltpu.VMEM((_KB, D), jnp.float32),
            pltpu.VMEM((_KB, D), jnp.float32),
            pltpu.VMEM_SHARED((_NP, D), jnp.float32),
            pltpu.SemaphoreType.DMA,
        ],
    )
    def sc_stage(nf_hbm, wa_hbm, src_hbm, dst_hbm, zeros_hbm, out_hbm,
                 src_v, dst_v, rows_v, wa_v, acc_sh, sem):
        cid = lax.axis_index("c")
        sid = lax.axis_index("s")
        wid = sid * _NC + cid

        pltpu.sync_copy(zeros_hbm.at[pl.ds(sid * _RPS, _RPS)],
                        acc_sh.at[pl.ds(sid * _RPS, _RPS)])
        plsc.subcore_barrier()

        def block_body(i, carry):
            base = wid * _EPW + i * _KB
            pltpu.sync_copy(src_hbm.at[pl.ds(base, _KB)], src_v)
            pltpu.sync_copy(dst_hbm.at[pl.ds(base, _KB)], dst_v)
            pltpu.sync_copy(wa_hbm.at[pl.ds(base, _KB)], wa_v)
            pltpu.async_copy(nf_hbm.at[src_v], rows_v, sem).wait()

            def mul_body(j, c2):
                for c in range(D // 16):
                    sl = pl.ds(c * 16, 16)
                    rows_v[j, sl] = rows_v[j, sl] * wa_v[j, sl]
                return c2

            lax.fori_loop(0, _KB, mul_body, 0)
            pltpu.sync_copy(rows_v, acc_sh.at[dst_v], add=True)
            return carry

        lax.fori_loop(0, _NB, block_body, 0)

        plsc.subcore_barrier()
        pltpu.sync_copy(acc_sh.at[pl.ds(sid * _RPS, _RPS)],
                        out_hbm.at[pl.ds(cid * _NP + sid * _RPS, _RPS)])

    return sc_stage


_sc_stage = _make_sc_stage()


def _final_body(p_ref, attr_ref, nsc_ref, w2_ref, w3_ref, out_ref):
    nf2 = (p_ref[0, :N, :] + p_ref[1, :N, :]) * INV_SQRT_NN
    a = attr_ref[...]
    conv = jnp.dot(nf2, w2_ref[...], preferred_element_type=jnp.float32) * (
        a * INV_SQRT_D)
    ang = jnp.sum(nf2 * w3_ref[...], axis=1, keepdims=True) * (
        a * (0.1 * INV_SQRT_D))
    out_ref[...] = jnp.cos(ang) * nsc_ref[...] + jnp.sin(ang) * conv


def _final_stage(parts, node_attr, nsc, w2, w3row):
    return pl.pallas_call(
        _final_body,
        out_shape=jax.ShapeDtypeStruct((N, D), jnp.float32),
    )(parts, node_attr, nsc, w2, w3row)


def kernel(node_input, node_attr, edge_src, edge_dst, edge_attr, edge_scalars,
           W_sc, W_lin1, fc_W1, fc_W2, W_lin2, W_lin3):
    src = edge_src.astype(jnp.int32)
    dst = edge_dst.astype(jnp.int32)
    nsc, nf = _node_stage(node_input, node_attr, W_sc[:, 0, :], W_lin1[:, 0, :])
    wa = _edge_stage(edge_scalars, edge_attr, fc_W1, fc_W2)
    zeros = jnp.zeros((_NP, D), jnp.float32)
    parts = _sc_stage(nf, wa, src, dst, zeros).reshape(_NC, _NP, D)
    return _final_stage(parts, node_attr, nsc, W_lin2[:, 0, :],
                        W_lin3[:, 0, :].T)


# speedup vs baseline: 2.3600x; 1.0413x over previous
import functools

import numpy as np
import jax
import jax.numpy as jnp
from jax import lax
from jax.experimental import pallas as pl
from jax.experimental.pallas import tpu as pltpu
from jax.experimental.pallas import tpu_sc as plsc

N, E, D, S, H = 10000, 320000, 128, 16, 64
INV_SQRT_D = 1.0 / np.sqrt(128.0)
INV_SQRT_S = 1.0 / np.sqrt(16.0)
INV_SQRT_H = 1.0 / np.sqrt(64.0)
INV_SQRT_NN = 1.0 / np.sqrt(32.0)


def _node_body(x_ref, attr_ref, wsc_ref, w1_ref, nsc_ref, nf_ref):
    x = x_ref[...]
    s = attr_ref[...] * INV_SQRT_D
    nsc_ref[...] = jnp.dot(x, wsc_ref[...], preferred_element_type=jnp.float32) * s
    nf_ref[...] = jnp.dot(x, w1_ref[...], preferred_element_type=jnp.float32) * s


def _node_stage(node_input, node_attr, wsc, w1):
    return pl.pallas_call(
        _node_body,
        out_shape=(
            jax.ShapeDtypeStruct((N, D), jnp.float32),
            jax.ShapeDtypeStruct((N, D), jnp.float32),
        ),
    )(node_input, node_attr, wsc, w1)


_BE = 3200
_EG = E // _BE


def _edge_body(es_ref, ea_ref, w1_ref, w2_ref, wa_ref):
    h = jnp.dot(es_ref[...], w1_ref[...], preferred_element_type=jnp.float32)
    h = jax.nn.silu(h * INV_SQRT_S)
    w = jnp.dot(h, w2_ref[...], preferred_element_type=jnp.float32) * INV_SQRT_H
    wa_ref[...] = w * ea_ref[...]


def _edge_stage(edge_scalars, edge_attr, fc_W1, fc_W2):
    return pl.pallas_call(
        _edge_body,
        grid=(_EG,),
        in_specs=[
            pl.BlockSpec((_BE, S), lambda i: (i, 0)),
            pl.BlockSpec((_BE, 1), lambda i: (i, 0)),
            pl.BlockSpec((S, H), lambda i: (0, 0)),
            pl.BlockSpec((H, D), lambda i: (0, 0)),
        ],
        out_specs=pl.BlockSpec((_BE, D), lambda i: (i, 0)),
        out_shape=jax.ShapeDtypeStruct((E, D), jnp.float32),
    )(edge_scalars, edge_attr, fc_W1, fc_W2)


_NC, _NS = 2, 16
_NW = _NC * _NS
_EPW = E // _NW
_KB = 80
_NB = _EPW // _KB
_RPS = 632
_NP = _RPS * _NS


def _make_sc_stage():
    mesh = plsc.VectorSubcoreMesh(core_axis_name="c", subcore_axis_name="s")

    @functools.partial(
        pl.kernel,
        out_type=jax.ShapeDtypeStruct((_NC * _NP, D), jnp.float32),
        mesh=mesh,
        scratch_types=[
            pltpu.VMEM((2, _KB), jnp.int32),
            pltpu.VMEM((2, _KB), jnp.int32),
            pltpu.VMEM((2, _KB, D), jnp.float32),
            pltpu.VMEM((2, _KB, D), jnp.float32),
            pltpu.VMEM_SHARED((_NP, D), jnp.float32),
            pltpu.SemaphoreType.DMA,
            pltpu.SemaphoreType.DMA,
            pltpu.SemaphoreType.DMA,
            pltpu.SemaphoreType.DMA,
            pltpu.SemaphoreType.DMA,
            pltpu.SemaphoreType.DMA,
            pltpu.SemaphoreType.DMA,
            pltpu.SemaphoreType.DMA,
        ],
    )
    def sc_stage(nf_hbm, wa_hbm, src_hbm, dst_hbm, zeros_hbm, out_hbm,
                 src_v, dst_v, rows_v, wa_v, acc_sh,
                 s_src0, s_src1, s_dst0, s_dst1, s_wa0, s_wa1, s_g0, s_g1):
        cid = lax.axis_index("c")
        sid = lax.axis_index("s")
        wid = sid * _NC + cid
        s_src = (s_src0, s_src1)
        s_dst = (s_dst0, s_dst1)
        s_wa = (s_wa0, s_wa1)
        s_g = (s_g0, s_g1)

        pltpu.sync_copy(zeros_hbm.at[pl.ds(sid * _RPS, _RPS)],
                        acc_sh.at[pl.ds(sid * _RPS, _RPS)])
        plsc.subcore_barrier()

        def issue_linear(i, b):
            base = wid * _EPW + i * _KB
            pltpu.async_copy(src_hbm.at[pl.ds(base, _KB)], src_v.at[b], s_src[b])
            pltpu.async_copy(dst_hbm.at[pl.ds(base, _KB)], dst_v.at[b], s_dst[b])
            pltpu.async_copy(wa_hbm.at[pl.ds(base, _KB)], wa_v.at[b], s_wa[b])

        def wait_src(b):
            pltpu.make_async_copy(src_hbm.at[pl.ds(0, _KB)], src_v.at[b],
                                  s_src[b]).wait()

        def issue_gather(b):
            pltpu.async_copy(nf_hbm.at[src_v.at[b]], rows_v.at[b], s_g[b])

        def process(i, b):
            pltpu.make_async_copy(nf_hbm.at[pl.ds(0, _KB)], rows_v.at[b],
                                  s_g[b]).wait()
            pltpu.make_async_copy(wa_hbm.at[pl.ds(0, _KB)], wa_v.at[b],
                                  s_wa[b]).wait()
            pltpu.make_async_copy(dst_hbm.at[pl.ds(0, _KB)], dst_v.at[b],
                                  s_dst[b]).wait()

            def mul_body(j, c2):
                for c in range(D // 16):
                    sl = pl.ds(c * 16, 16)
                    rows_v[b, j, sl] = rows_v[b, j, sl] * wa_v[b, j, sl]
                return c2

            lax.fori_loop(0, _KB, mul_body, 0, unroll=4)
            pltpu.sync_copy(rows_v.at[b], acc_sh.at[dst_v.at[b]], add=True)

        issue_linear(0, 0)
        wait_src(0)
        issue_gather(0)
        issue_linear(1, 1)

        def outer(i, carry):
            for b in (0, 1):
                ib = i + b
                q = 1 - b

                @pl.when(ib + 1 < _NB)
                def _():
                    wait_src(q)
                    issue_gather(q)

                @pl.when(ib < _NB)
                def _():
                    process(ib, b)

                @pl.when(ib + 2 < _NB)
                def _():
                    issue_linear(ib + 2, b)
            return carry

        lax.fori_loop(0, (_NB + 1) // 2, lambda t, c: outer(t * 2, c), 0)

        plsc.subcore_barrier()
        pltpu.sync_copy(acc_sh.at[pl.ds(sid * _RPS, _RPS)],
                        out_hbm.at[pl.ds(cid * _NP + sid * _RPS, _RPS)])

    return sc_stage


_sc_stage = _make_sc_stage()


def _final_body(p_ref, attr_ref, nsc_ref, w2_ref, w3_ref, out_ref):
    nf2 = (p_ref[0, :N, :] + p_ref[1, :N, :]) * INV_SQRT_NN
    a = attr_ref[...]
    conv = jnp.dot(nf2, w2_ref[...], preferred_element_type=jnp.float32) * (
        a * INV_SQRT_D)
    ang = jnp.sum(nf2 * w3_ref[...], axis=1, keepdims=True) * (
        a * (0.1 * INV_SQRT_D))
    out_ref[...] = jnp.cos(ang) * nsc_ref[...] + jnp.sin(ang) * conv


def _final_stage(parts, node_attr, nsc, w2, w3row):
    return pl.pallas_call(
        _final_body,
        out_shape=jax.ShapeDtypeStruct((N, D), jnp.float32),
    )(parts, node_attr, nsc, w2, w3row)


def kernel(node_input, node_attr, edge_src, edge_dst, edge_attr, edge_scalars,
           W_sc, W_lin1, fc_W1, fc_W2, W_lin2, W_lin3):
    src = edge_src.astype(jnp.int32)
    dst = edge_dst.astype(jnp.int32)
    nsc, nf = _node_stage(node_input, node_attr, W_sc[:, 0, :], W_lin1[:, 0, :])
    wa = _edge_stage(edge_scalars, edge_attr, fc_W1, fc_W2)
    zeros = jnp.zeros((_NP, D), jnp.float32)
    parts = _sc_stage(nf, wa, src, dst, zeros).reshape(_NC, _NP, D)
    return _final_stage(parts, node_attr, nsc, W_lin2[:, 0, :],
                        W_lin3[:, 0, :].T)


# speedup vs baseline: 2.9398x; 1.2457x over previous
import functools

import numpy as np
import jax
import jax.numpy as jnp
from jax import lax
from jax.experimental import pallas as pl
from jax.experimental.pallas import tpu as pltpu
from jax.experimental.pallas import tpu_sc as plsc

N, E, D, S, H = 10000, 320000, 128, 16, 64
INV_SQRT_D = 1.0 / np.sqrt(128.0)
INV_SQRT_S = 1.0 / np.sqrt(16.0)
INV_SQRT_H = 1.0 / np.sqrt(64.0)
INV_SQRT_NN = 1.0 / np.sqrt(32.0)


def _node_body(x_ref, attr_ref, wsc_ref, w1_ref, nsc_ref, nf_ref):
    x = x_ref[...]
    s = attr_ref[...] * INV_SQRT_D
    nsc_ref[...] = jnp.dot(x, wsc_ref[...], preferred_element_type=jnp.float32) * s
    nf_ref[...] = jnp.dot(x, w1_ref[...], preferred_element_type=jnp.float32) * s


def _node_stage(node_input, node_attr, wsc, w1):
    return pl.pallas_call(
        _node_body,
        out_shape=(
            jax.ShapeDtypeStruct((N, D), jnp.float32),
            jax.ShapeDtypeStruct((N, D), jnp.float32),
        ),
    )(node_input, node_attr, wsc, w1)


_BE = 3200
_EG = E // _BE


def _edge_body(es_ref, ea_ref, w1_ref, w2_ref, wa_ref):
    h = jnp.dot(es_ref[...], w1_ref[...], preferred_element_type=jnp.float32)
    h = jax.nn.silu(h * INV_SQRT_S)
    w = jnp.dot(h, w2_ref[...], preferred_element_type=jnp.float32) * INV_SQRT_H
    wa_ref[...] = w * ea_ref[...]


def _edge_stage(edge_scalars, edge_attr, fc_W1, fc_W2):
    return pl.pallas_call(
        _edge_body,
        grid=(_EG,),
        in_specs=[
            pl.BlockSpec((_BE, S), lambda i: (i, 0)),
            pl.BlockSpec((_BE, 1), lambda i: (i, 0)),
            pl.BlockSpec((S, H), lambda i: (0, 0)),
            pl.BlockSpec((H, D), lambda i: (0, 0)),
        ],
        out_specs=pl.BlockSpec((_BE, D), lambda i: (i, 0)),
        out_shape=jax.ShapeDtypeStruct((E, D), jnp.float32),
    )(edge_scalars, edge_attr, fc_W1, fc_W2)


_NC, _NS = 2, 16
_NW = _NC * _NS
_EPW = E // _NW
_KB = 40
_NB = _EPW // _KB
_RPS = 632
_NP = _RPS * _NS


def _make_sc_stage():
    mesh = plsc.VectorSubcoreMesh(core_axis_name="c", subcore_axis_name="s")

    @functools.partial(
        pl.kernel,
        out_type=jax.ShapeDtypeStruct((_NC * _NP, D), jnp.float32),
        mesh=mesh,
        scratch_types=[
            pltpu.VMEM((2, _KB), jnp.int32),
            pltpu.VMEM((2, _KB), jnp.int32),
            pltpu.VMEM((2, _KB, D), jnp.float32),
            pltpu.VMEM((2, _KB, D), jnp.float32),
            pltpu.VMEM((2, _KB, D), jnp.float32),
            pltpu.VMEM_SHARED((_NP, D), jnp.float32),
            pltpu.SemaphoreType.DMA,
            pltpu.SemaphoreType.DMA,
            pltpu.SemaphoreType.DMA,
            pltpu.SemaphoreType.DMA,
            pltpu.SemaphoreType.DMA,
            pltpu.SemaphoreType.DMA,
            pltpu.SemaphoreType.DMA,
            pltpu.SemaphoreType.DMA,
        ],
    )
    def sc_stage(nf_hbm, wa_hbm, src_hbm, dst_hbm, zeros_hbm, out_hbm,
                 src_v, dst_v, rows_v, wa_v, prod_v, acc_sh,
                 s_src0, s_src1, s_dst0, s_dst1, s_wa0, s_wa1, s_g0, s_g1):
        cid = lax.axis_index("c")
        sid = lax.axis_index("s")
        wid = sid * _NC + cid
        s_src = (s_src0, s_src1)
        s_dst = (s_dst0, s_dst1)
        s_wa = (s_wa0, s_wa1)
        s_g = (s_g0, s_g1)

        pltpu.sync_copy(zeros_hbm.at[pl.ds(sid * _RPS, _RPS)],
                        acc_sh.at[pl.ds(sid * _RPS, _RPS)])
        plsc.subcore_barrier()

        def issue_linear(i, b):
            base = wid * _EPW + i * _KB
            pltpu.async_copy(src_hbm.at[pl.ds(base, _KB)], src_v.at[b], s_src[b])
            pltpu.async_copy(dst_hbm.at[pl.ds(base, _KB)], dst_v.at[b], s_dst[b])
            pltpu.async_copy(wa_hbm.at[pl.ds(base, _KB)], wa_v.at[b], s_wa[b])

        def wait_src(b):
            pltpu.make_async_copy(src_hbm.at[pl.ds(0, _KB)], src_v.at[b],
                                  s_src[b]).wait()

        def issue_gather(b):
            pltpu.async_copy(nf_hbm.at[src_v.at[b]], rows_v.at[b], s_g[b])

        def process(i, b):
            pltpu.make_async_copy(nf_hbm.at[pl.ds(0, _KB)], rows_v.at[b],
                                  s_g[b]).wait()
            pltpu.make_async_copy(wa_hbm.at[pl.ds(0, _KB)], wa_v.at[b],
                                  s_wa[b]).wait()
            pltpu.make_async_copy(dst_hbm.at[pl.ds(0, _KB)], dst_v.at[b],
                                  s_dst[b]).wait()

            def mul_body(j, c2):
                r = [rows_v[b, j, pl.ds(c * 16, 16)] for c in range(D // 16)]
                w = [wa_v[b, j, pl.ds(c * 16, 16)] for c in range(D // 16)]
                for c in range(D // 16):
                    prod_v[b, j, pl.ds(c * 16, 16)] = r[c] * w[c]
                return c2

            lax.fori_loop(0, _KB, mul_body, 0, unroll=4)
            pltpu.sync_copy(prod_v.at[b], acc_sh.at[dst_v.at[b]], add=True)

        issue_linear(0, 0)
        wait_src(0)
        issue_gather(0)
        issue_linear(1, 1)

        def outer(i, carry):
            for b in (0, 1):
                ib = i + b
                q = 1 - b

                @pl.when(ib + 1 < _NB)
                def _():
                    wait_src(q)
                    issue_gather(q)

                @pl.when(ib < _NB)
                def _():
                    process(ib, b)

                @pl.when(ib + 2 < _NB)
                def _():
                    issue_linear(ib + 2, b)
            return carry

        lax.fori_loop(0, (_NB + 1) // 2, lambda t, c: outer(t * 2, c), 0)

        plsc.subcore_barrier()
        pltpu.sync_copy(acc_sh.at[pl.ds(sid * _RPS, _RPS)],
                        out_hbm.at[pl.ds(cid * _NP + sid * _RPS, _RPS)])

    return sc_stage


_sc_stage = _make_sc_stage()


def _final_body(p_ref, attr_ref, nsc_ref, w2_ref, w3_ref, out_ref):
    nf2 = (p_ref[0, :N, :] + p_ref[1, :N, :]) * INV_SQRT_NN
    a = attr_ref[...]
    conv = jnp.dot(nf2, w2_ref[...], preferred_element_type=jnp.float32) * (
        a * INV_SQRT_D)
    ang = jnp.sum(nf2 * w3_ref[...], axis=1, keepdims=True) * (
        a * (0.1 * INV_SQRT_D))
    out_ref[...] = jnp.cos(ang) * nsc_ref[...] + jnp.sin(ang) * conv


def _final_stage(parts, node_attr, nsc, w2, w3row):
    return pl.pallas_call(
        _final_body,
        out_shape=jax.ShapeDtypeStruct((N, D), jnp.float32),
    )(parts, node_attr, nsc, w2, w3row)


def kernel(node_input, node_attr, edge_src, edge_dst, edge_attr, edge_scalars,
           W_sc, W_lin1, fc_W1, fc_W2, W_lin2, W_lin3):
    src = edge_src.astype(jnp.int32)
    dst = edge_dst.astype(jnp.int32)
    nsc, nf = _node_stage(node_input, node_attr, W_sc[:, 0, :], W_lin1[:, 0, :])
    wa = _edge_stage(edge_scalars, edge_attr, fc_W1, fc_W2)
    zeros = jnp.zeros((_NP, D), jnp.float32)
    parts = _sc_stage(nf, wa, src, dst, zeros).reshape(_NC, _NP, D)
    return _final_stage(parts, node_attr, nsc, W_lin2[:, 0, :],
                        W_lin3[:, 0, :].T)


# speedup vs baseline: 3.1984x; 1.0880x over previous
import functools

import numpy as np
import jax
import jax.numpy as jnp
from jax import lax
from jax.experimental import pallas as pl
from jax.experimental.pallas import tpu as pltpu
from jax.experimental.pallas import tpu_sc as plsc

N, E, D, S, H = 10000, 320000, 128, 16, 64
INV_SQRT_D = 1.0 / np.sqrt(128.0)
INV_SQRT_S = 1.0 / np.sqrt(16.0)
INV_SQRT_H = 1.0 / np.sqrt(64.0)
INV_SQRT_NN = 1.0 / np.sqrt(32.0)


def _node_body(x_ref, attr_ref, wsc_ref, w1_ref, nsc_ref, nf_ref):
    x = x_ref[...]
    s = attr_ref[...] * INV_SQRT_D
    nsc_ref[...] = jnp.dot(x, wsc_ref[...], preferred_element_type=jnp.float32) * s
    nf_ref[...] = jnp.dot(x, w1_ref[...], preferred_element_type=jnp.float32) * s


def _node_stage(node_input, node_attr, wsc, w1):
    return pl.pallas_call(
        _node_body,
        out_shape=(
            jax.ShapeDtypeStruct((N, D), jnp.float32),
            jax.ShapeDtypeStruct((N, D), jnp.float32),
        ),
    )(node_input, node_attr, wsc, w1)


_BE = 3200
_EG = E // _BE


def _edge_body(es_ref, ea_ref, w1_ref, w2_ref, wa_ref):
    h = jnp.dot(es_ref[...], w1_ref[...], preferred_element_type=jnp.float32)
    h = jax.nn.silu(h * INV_SQRT_S)
    w = jnp.dot(h, w2_ref[...], preferred_element_type=jnp.float32) * INV_SQRT_H
    wa_ref[...] = w * ea_ref[...]


def _edge_stage(edge_scalars, edge_attr, fc_W1, fc_W2):
    return pl.pallas_call(
        _edge_body,
        grid=(_EG,),
        in_specs=[
            pl.BlockSpec((_BE, S), lambda i: (i, 0)),
            pl.BlockSpec((_BE, 1), lambda i: (i, 0)),
            pl.BlockSpec((S, H), lambda i: (0, 0)),
            pl.BlockSpec((H, D), lambda i: (0, 0)),
        ],
        out_specs=pl.BlockSpec((_BE, D), lambda i: (i, 0)),
        out_shape=jax.ShapeDtypeStruct((E, D), jnp.float32),
    )(edge_scalars, edge_attr, fc_W1, fc_W2)


_NC, _NS = 2, 16
_NW = _NC * _NS
_EPW = E // _NW
_KB = 40
_NB = _EPW // _KB
_RPS = 632
_NP = _RPS * _NS


def _make_sc_stage():
    mesh = plsc.VectorSubcoreMesh(core_axis_name="c", subcore_axis_name="s")

    @functools.partial(
        pl.kernel,
        out_type=jax.ShapeDtypeStruct((_NC * _NP, D), jnp.float32),
        mesh=mesh,
        scratch_types=[
            pltpu.VMEM((2, _KB), jnp.int32),
            pltpu.VMEM((2, _KB), jnp.int32),
            pltpu.VMEM((2, _KB, D), jnp.float32),
            pltpu.VMEM((2, _KB, D), jnp.float32),
            pltpu.VMEM((2, _KB, D), jnp.float32),
            pltpu.VMEM_SHARED((_NP, D), jnp.float32),
            pltpu.SemaphoreType.DMA,
            pltpu.SemaphoreType.DMA,
            pltpu.SemaphoreType.DMA,
            pltpu.SemaphoreType.DMA,
            pltpu.SemaphoreType.DMA,
            pltpu.SemaphoreType.DMA,
            pltpu.SemaphoreType.DMA,
            pltpu.SemaphoreType.DMA,
            pltpu.SemaphoreType.DMA,
            pltpu.SemaphoreType.DMA,
        ],
    )
    def sc_stage(nf_hbm, wa_hbm, src_hbm, dst_hbm, zeros_hbm, out_hbm,
                 src_v, dst_v, rows_v, wa_v, prod_v, acc_sh,
                 s_src0, s_src1, s_dst0, s_dst1, s_wa0, s_wa1, s_g0, s_g1,
                 s_sc0, s_sc1):
        cid = lax.axis_index("c")
        sid = lax.axis_index("s")
        wid = sid * _NC + cid
        s_src = (s_src0, s_src1)
        s_dst = (s_dst0, s_dst1)
        s_wa = (s_wa0, s_wa1)
        s_g = (s_g0, s_g1)
        s_sc = (s_sc0, s_sc1)

        pltpu.sync_copy(zeros_hbm.at[pl.ds(sid * _RPS, _RPS)],
                        acc_sh.at[pl.ds(sid * _RPS, _RPS)])
        plsc.subcore_barrier()

        def issue_linear(i, b):
            base = wid * _EPW + i * _KB
            pltpu.async_copy(src_hbm.at[pl.ds(base, _KB)], src_v.at[b], s_src[b])
            pltpu.async_copy(wa_hbm.at[pl.ds(base, _KB)], wa_v.at[b], s_wa[b])

        def wait_src(b):
            pltpu.make_async_copy(src_hbm.at[pl.ds(0, _KB)], src_v.at[b],
                                  s_src[b]).wait()

        def issue_gather(b):
            pltpu.async_copy(nf_hbm.at[src_v.at[b]], rows_v.at[b], s_g[b])

        def wait_scatter(b):
            pltpu.make_async_copy(prod_v.at[b], acc_sh.at[pl.ds(0, _KB)],
                                  s_sc[b]).wait()

        def process(i, b):
            @pl.when(i >= 2)
            def _():
                wait_scatter(b)

            base = wid * _EPW + i * _KB
            pltpu.async_copy(dst_hbm.at[pl.ds(base, _KB)], dst_v.at[b],
                             s_dst[b])

            pltpu.make_async_copy(nf_hbm.at[pl.ds(0, _KB)], rows_v.at[b],
                                  s_g[b]).wait()
            pltpu.make_async_copy(wa_hbm.at[pl.ds(0, _KB)], wa_v.at[b],
                                  s_wa[b]).wait()

            def mul_body(j, c2):
                r = [rows_v[b, j, pl.ds(c * 16, 16)] for c in range(D // 16)]
                w = [wa_v[b, j, pl.ds(c * 16, 16)] for c in range(D // 16)]
                for c in range(D // 16):
                    prod_v[b, j, pl.ds(c * 16, 16)] = r[c] * w[c]
                return c2

            lax.fori_loop(0, _KB, mul_body, 0, unroll=4)
            pltpu.make_async_copy(dst_hbm.at[pl.ds(0, _KB)], dst_v.at[b],
                                  s_dst[b]).wait()
            pltpu.async_copy(prod_v.at[b], acc_sh.at[dst_v.at[b]], s_sc[b],
                             add=True)

        issue_linear(0, 0)
        wait_src(0)
        issue_gather(0)
        issue_linear(1, 1)

        def outer(i, carry):
            for b in (0, 1):
                ib = i + b
                q = 1 - b

                @pl.when(ib + 1 < _NB)
                def _():
                    wait_src(q)
                    issue_gather(q)

                @pl.when(ib < _NB)
                def _():
                    process(ib, b)

                @pl.when(ib + 2 < _NB)
                def _():
                    issue_linear(ib + 2, b)
            return carry

        lax.fori_loop(0, (_NB + 1) // 2, lambda t, c: outer(t * 2, c), 0)
        wait_scatter(0)
        wait_scatter(1)

        plsc.subcore_barrier()
        pltpu.sync_copy(acc_sh.at[pl.ds(sid * _RPS, _RPS)],
                        out_hbm.at[pl.ds(cid * _NP + sid * _RPS, _RPS)])

    return sc_stage


_sc_stage = _make_sc_stage()


def _final_body(p_ref, attr_ref, nsc_ref, w2_ref, w3_ref, out_ref):
    nf2 = (p_ref[0, :N, :] + p_ref[1, :N, :]) * INV_SQRT_NN
    a = attr_ref[...]
    conv = jnp.dot(nf2, w2_ref[...], preferred_element_type=jnp.float32) * (
        a * INV_SQRT_D)
    ang = jnp.sum(nf2 * w3_ref[...], axis=1, keepdims=True) * (
        a * (0.1 * INV_SQRT_D))
    out_ref[...] = jnp.cos(ang) * nsc_ref[...] + jnp.sin(ang) * conv


def _final_stage(parts, node_attr, nsc, w2, w3row):
    return pl.pallas_call(
        _final_body,
        out_shape=jax.ShapeDtypeStruct((N, D), jnp.float32),
    )(parts, node_attr, nsc, w2, w3row)


def kernel(node_input, node_attr, edge_src, edge_dst, edge_attr, edge_scalars,
           W_sc, W_lin1, fc_W1, fc_W2, W_lin2, W_lin3):
    src = edge_src.astype(jnp.int32)
    dst = edge_dst.astype(jnp.int32)
    nsc, nf = _node_stage(node_input, node_attr, W_sc[:, 0, :], W_lin1[:, 0, :])
    wa = _edge_stage(edge_scalars, edge_attr, fc_W1, fc_W2)
    zeros = jnp.zeros((_NP, D), jnp.float32)
    parts = _sc_stage(nf, wa, src, dst, zeros).reshape(_NC, _NP, D)
    return _final_stage(parts, node_attr, nsc, W_lin2[:, 0, :],
                        W_lin3[:, 0, :].T)


# speedup vs baseline: 3.2165x; 1.0056x over previous
import functools

import numpy as np
import jax
import jax.numpy as jnp
from jax import lax
from jax.experimental import pallas as pl
from jax.experimental.pallas import tpu as pltpu
from jax.experimental.pallas import tpu_sc as plsc

N, E, D, S, H = 10000, 320000, 128, 16, 64
INV_SQRT_D = 1.0 / np.sqrt(128.0)
INV_SQRT_S = 1.0 / np.sqrt(16.0)
INV_SQRT_H = 1.0 / np.sqrt(64.0)
INV_SQRT_NN = 1.0 / np.sqrt(32.0)


def _node_body(x_ref, attr_ref, wsc_ref, w1_ref, nsc_ref, nf_ref):
    x = x_ref[...]
    s = attr_ref[...] * INV_SQRT_D
    nsc_ref[...] = jnp.dot(x, wsc_ref[...], preferred_element_type=jnp.float32) * s
    nf_ref[...] = jnp.dot(x, w1_ref[...], preferred_element_type=jnp.float32) * s


def _node_stage(node_input, node_attr, wsc, w1):
    return pl.pallas_call(
        _node_body,
        out_shape=(
            jax.ShapeDtypeStruct((N, D), jnp.float32),
            jax.ShapeDtypeStruct((N, D), jnp.float32),
        ),
    )(node_input, node_attr, wsc, w1)


_BE = 3200
_EG = E // _BE


def _edge_body(es_ref, ea_ref, w1_ref, w2_ref, wa_ref):
    h = jnp.dot(es_ref[...], w1_ref[...], preferred_element_type=jnp.float32)
    h = jax.nn.silu(h * INV_SQRT_S)
    w = jnp.dot(h, w2_ref[...], preferred_element_type=jnp.float32) * INV_SQRT_H
    w = w * ea_ref[...]
    a = jax.lax.bitcast_convert_type(w[:, :D // 2], jnp.int32)
    b2 = jax.lax.bitcast_convert_type(w[:, D // 2:], jnp.int32)
    lo = jax.lax.shift_right_logical(a + 0x8000, 16)
    hi = jnp.bitwise_and(b2 + 0x8000, -65536)
    wa_ref[...] = jnp.bitwise_or(lo, hi)


def _edge_stage(edge_scalars, edge_attr, fc_W1, fc_W2):
    return pl.pallas_call(
        _edge_body,
        grid=(_EG,),
        in_specs=[
            pl.BlockSpec((_BE, S), lambda i: (i, 0)),
            pl.BlockSpec((_BE, 1), lambda i: (i, 0)),
            pl.BlockSpec((S, H), lambda i: (0, 0)),
            pl.BlockSpec((H, D), lambda i: (0, 0)),
        ],
        out_specs=pl.BlockSpec((_BE, D // 2), lambda i: (i, 0)),
        out_shape=jax.ShapeDtypeStruct((E, D // 2), jnp.int32),
    )(edge_scalars, edge_attr, fc_W1, fc_W2)


_NC, _NS = 2, 16
_NW = _NC * _NS
_EPW = E // _NW
_KB = 40
_NB = _EPW // _KB
_RPS = 632
_NP = _RPS * _NS


def _make_sc_stage():
    mesh = plsc.VectorSubcoreMesh(core_axis_name="c", subcore_axis_name="s")

    @functools.partial(
        pl.kernel,
        out_type=jax.ShapeDtypeStruct((_NC * _NP, D), jnp.float32),
        mesh=mesh,
        scratch_types=[
            pltpu.VMEM((2, _KB), jnp.int32),
            pltpu.VMEM((2, _KB), jnp.int32),
            pltpu.VMEM((2, _KB, D), jnp.float32),
            pltpu.VMEM((2, _KB, D // 2), jnp.int32),
            pltpu.VMEM((2, _KB, D), jnp.float32),
            pltpu.VMEM_SHARED((_NP, D), jnp.float32),
            pltpu.SemaphoreType.DMA,
            pltpu.SemaphoreType.DMA,
            pltpu.SemaphoreType.DMA,
            pltpu.SemaphoreType.DMA,
            pltpu.SemaphoreType.DMA,
            pltpu.SemaphoreType.DMA,
            pltpu.SemaphoreType.DMA,
            pltpu.SemaphoreType.DMA,
            pltpu.SemaphoreType.DMA,
            pltpu.SemaphoreType.DMA,
        ],
    )
    def sc_stage(nf_hbm, wa_hbm, src_hbm, dst_hbm, zeros_hbm, out_hbm,
                 src_v, dst_v, rows_v, wa_v, prod_v, acc_sh,
                 s_src0, s_src1, s_dst0, s_dst1, s_wa0, s_wa1, s_g0, s_g1,
                 s_sc0, s_sc1):
        cid = lax.axis_index("c")
        sid = lax.axis_index("s")
        wid = sid * _NC + cid
        s_src = (s_src0, s_src1)
        s_dst = (s_dst0, s_dst1)
        s_wa = (s_wa0, s_wa1)
        s_g = (s_g0, s_g1)
        s_sc = (s_sc0, s_sc1)

        pltpu.sync_copy(zeros_hbm.at[pl.ds(sid * _RPS, _RPS)],
                        acc_sh.at[pl.ds(sid * _RPS, _RPS)])
        plsc.subcore_barrier()

        def issue_linear(i, b):
            base = wid * _EPW + i * _KB
            pltpu.async_copy(src_hbm.at[pl.ds(base, _KB)], src_v.at[b], s_src[b])
            pltpu.async_copy(wa_hbm.at[pl.ds(base, _KB)], wa_v.at[b], s_wa[b])

        def wait_src(b):
            pltpu.make_async_copy(src_hbm.at[pl.ds(0, _KB)], src_v.at[b],
                                  s_src[b]).wait()

        def issue_gather(b):
            pltpu.async_copy(nf_hbm.at[src_v.at[b]], rows_v.at[b], s_g[b])

        def wait_scatter(b):
            pltpu.make_async_copy(prod_v.at[b], acc_sh.at[pl.ds(0, _KB)],
                                  s_sc[b]).wait()

        def process(i, b):
            @pl.when(i >= 2)
            def _():
                wait_scatter(b)

            base = wid * _EPW + i * _KB
            pltpu.async_copy(dst_hbm.at[pl.ds(base, _KB)], dst_v.at[b],
                             s_dst[b])

            pltpu.make_async_copy(nf_hbm.at[pl.ds(0, _KB)], rows_v.at[b],
                                  s_g[b]).wait()
            pltpu.make_async_copy(wa_hbm.at[pl.ds(0, _KB)], wa_v.at[b],
                                  s_wa[b]).wait()

            def mul_body(j, c2):
                r = [rows_v[b, j, pl.ds(c * 16, 16)] for c in range(D // 16)]
                ww = [wa_v[b, j, pl.ds(g * 16, 16)] for g in range(D // 32)]
                c16 = jnp.full((16,), 16, jnp.int32)
                cmask = jnp.full((16,), -65536, jnp.int32)
                for g in range(D // 32):
                    wlo = jax.lax.bitcast_convert_type(
                        jnp.left_shift(ww[g], c16), jnp.float32)
                    whi = jax.lax.bitcast_convert_type(
                        jnp.bitwise_and(ww[g], cmask), jnp.float32)
                    prod_v[b, j, pl.ds(g * 16, 16)] = r[g] * wlo
                    prod_v[b, j, pl.ds((g + 4) * 16, 16)] = r[g + 4] * whi
                return c2

            lax.fori_loop(0, _KB, mul_body, 0, unroll=4)
            pltpu.make_async_copy(dst_hbm.at[pl.ds(0, _KB)], dst_v.at[b],
                                  s_dst[b]).wait()
            pltpu.async_copy(prod_v.at[b], acc_sh.at[dst_v.at[b]], s_sc[b],
                             add=True)

        issue_linear(0, 0)
        wait_src(0)
        issue_gather(0)
        issue_linear(1, 1)

        def outer(i, carry):
            for b in (0, 1):
                ib = i + b
                q = 1 - b

                @pl.when(ib + 1 < _NB)
                def _():
                    wait_src(q)
                    issue_gather(q)

                @pl.when(ib < _NB)
                def _():
                    process(ib, b)

                @pl.when(ib + 2 < _NB)
                def _():
                    issue_linear(ib + 2, b)
            return carry

        lax.fori_loop(0, (_NB + 1) // 2, lambda t, c: outer(t * 2, c), 0)
        wait_scatter(0)
        wait_scatter(1)

        plsc.subcore_barrier()
        pltpu.sync_copy(acc_sh.at[pl.ds(sid * _RPS, _RPS)],
                        out_hbm.at[pl.ds(cid * _NP + sid * _RPS, _RPS)])

    return sc_stage


_sc_stage = _make_sc_stage()


def _final_body(p_ref, attr_ref, nsc_ref, w2_ref, w3_ref, out_ref):
    nf2 = (p_ref[0, :N, :] + p_ref[1, :N, :]) * INV_SQRT_NN
    a = attr_ref[...]
    conv = jnp.dot(nf2, w2_ref[...], preferred_element_type=jnp.float32) * (
        a * INV_SQRT_D)
    ang = jnp.sum(nf2 * w3_ref[...], axis=1, keepdims=True) * (
        a * (0.1 * INV_SQRT_D))
    out_ref[...] = jnp.cos(ang) * nsc_ref[...] + jnp.sin(ang) * conv


def _final_stage(parts, node_attr, nsc, w2, w3row):
    return pl.pallas_call(
        _final_body,
        out_shape=jax.ShapeDtypeStruct((N, D), jnp.float32),
    )(parts, node_attr, nsc, w2, w3row)


def kernel(node_input, node_attr, edge_src, edge_dst, edge_attr, edge_scalars,
           W_sc, W_lin1, fc_W1, fc_W2, W_lin2, W_lin3):
    src = edge_src.astype(jnp.int32)
    dst = edge_dst.astype(jnp.int32)
    nsc, nf = _node_stage(node_input, node_attr, W_sc[:, 0, :], W_lin1[:, 0, :])
    wa = _edge_stage(edge_scalars, edge_attr, fc_W1, fc_W2)
    zeros = jnp.zeros((_NP, D), jnp.float32)
    parts = _sc_stage(nf, wa, src, dst, zeros).reshape(_NC, _NP, D)
    return _final_stage(parts, node_attr, nsc, W_lin2[:, 0, :],
                        W_lin3[:, 0, :].T)
